# baseline (speedup 1.0000x reference)
import jax
import jax.numpy as jnp
from jax import lax
from jax.experimental import pallas as pl
from jax.experimental.pallas import tpu as pltpu

_DN = ('NCHW', 'OIHW', 'NCHW')


def _conv(x, w, b, stride=1):
    y = lax.conv_general_dilated(x, w, (stride, stride), ((1, 1), (1, 1)),
                                 dimension_numbers=_DN)
    return y + b[None, :, None, None]


def _deconv(x, w, b):
    y = lax.conv_general_dilated(x, w, (1, 1), ((1, 2), (1, 2)),
                                 lhs_dilation=(2, 2), dimension_numbers=_DN)
    return y + b[None, :, None, None]


def _gdn(x, beta, gamma, inverse=False):
    norm = jnp.sqrt(jnp.einsum('bihw,oi->bohw', x * x, gamma)
                    + beta[None, :, None, None])
    return x * norm if inverse else x / norm


def _resblock(x, w1, b1, w2, b2):
    h = jax.nn.relu(_conv(x, w1, b1, 1))
    return _conv(h, w2, b2, 1) + x


_PAD = 8
_NSTRIP = 8
_UNROLL = 8


def _shift_x(v, ox, w):
    if ox == 0:
        return v
    z = jnp.zeros((v.shape[0], 1), jnp.float32)
    if ox == 1:
        return jnp.concatenate([v[:, 1:], z], axis=1)
    return jnp.concatenate([z, v[:, :w - 1]], axis=1)


def _tail_kernel(t_ref, w1_ref, b1_ref, w2_ref, b2_ref,
                 rec_ref, ent_ref, pad_ref, h_ref):
    himg = rec_ref.shape[2]
    wimg = rec_ref.shape[3]
    strip = himg // _NSTRIP

    zband = jnp.zeros((_PAD, wimg), jnp.float32)
    for c in range(3):
        pad_ref[c, 0:_PAD, :] = zband
        pad_ref[c, _PAD + himg:, :] = zband
        pad_ref[c, _PAD:_PAD + himg, :] = t_ref[0, c]
        h_ref[c, 0:_PAD, :] = zband
        h_ref[c, _PAD + himg:, :] = zband

    def conv_strip(y0, src_read, wref, bref):
        outs = [jnp.full((strip, wimg), bref[co], jnp.float32)
                for co in range(3)]
        for ci in range(3):
            big = src_read(ci, y0)
            for dy in range(3):
                r = big[_PAD + dy - 1:_PAD + dy - 1 + strip]
                for dx in range(3):
                    t = _shift_x(r, dx - 1, wimg)
                    for co in range(3):
                        k = ((co * 3 + ci) * 3 + dy) * 3 + dx
                        outs[co] = outs[co] + wref[k] * t
        return outs

    def t_read(ci, y0):
        return pad_ref[ci, pl.ds(y0, strip + 2 * _PAD), :]

    def h_read(ci, y0):
        return h_ref[ci, pl.ds(y0, strip + 2 * _PAD), :]

    def conv1_body(s, carry):
        y0 = pl.multiple_of(s * strip, 8)
        outs = conv_strip(y0, t_read, w1_ref, b1_ref)
        for co in range(3):
            h_ref[co, pl.ds(y0 + _PAD, strip), :] = (
                jnp.maximum(outs[co], 0.0))
        return carry

    lax.fori_loop(0, _NSTRIP, conv1_body, 0)

    def conv2_body(s, carry):
        mn, mx = carry
        y0 = pl.multiple_of(s * strip, 8)
        outs = conv_strip(y0, h_read, w2_ref, b2_ref)
        for co in range(3):
            rec = jax.nn.sigmoid(
                outs[co] + t_ref[0, co, pl.ds(y0, strip), :])
            rec_ref[0, co, pl.ds(y0, strip), :] = rec
            mn = jnp.minimum(mn, jnp.min(rec))
            mx = jnp.maximum(mx, jnp.max(rec))
        return mn, mx

    mn, mx = lax.fori_loop(0, _NSTRIP, conv2_body,
                           (jnp.float32(jnp.inf), jnp.float32(-jnp.inf)))
    scale = jnp.where(mx > mn, 256.0 / (mx - mn), 0.0)

    iot = lax.broadcasted_iota(jnp.int32, (16, 8, wimg), 0)

    def one(c, j):
        chunk = rec_ref[0, c, pl.ds(j * 8, 8), :]
        idx = jnp.clip(jnp.floor((chunk - mn) * scale), 0.0, 255.0
                       ).astype(jnp.int32)
        hi = idx >> 4
        lo = idx & 15
        H = jnp.where(iot == hi[None], 1.0, 0.0).astype(jnp.bfloat16)
        L = jnp.where(iot == lo[None], 1.0, 0.0).astype(jnp.bfloat16)
        return lax.dot_general(H.reshape(128, wimg), L.reshape(128, wimg),
                               (((1,), (1,)), ((), ())),
                               preferred_element_type=jnp.float32)

    accs = tuple(jnp.zeros((128, 128), jnp.float32) for _ in range(_UNROLL))
    nchunk = himg // 8
    for c in range(3):
        def body(i, carry):
            return tuple(a + one(c, i * _UNROLL + u)
                         for u, a in enumerate(carry))
        accs = lax.fori_loop(0, nchunk // _UNROLL, body, accs)
    big = accs[0]
    for a in accs[1:]:
        big = big + a

    i_sub = lax.broadcasted_iota(jnp.int32, (128, 128), 0) & 7
    j_sub = lax.broadcasted_iota(jnp.int32, (128, 128), 1) & 7
    masked = jnp.where(i_sub == j_sub, big, 0.0)
    s = jnp.sum(masked.reshape(16, 8, 128), axis=1)
    gj = lax.broadcasted_iota(jnp.int32, (128, 16), 0) >> 3
    gl = lax.broadcasted_iota(jnp.int32, (128, 16), 1)
    G = jnp.where(gj == gl, 1.0, 0.0)
    hist2 = lax.dot_general(s, G, (((1,), (0,)), ((), ())),
                            preferred_element_type=jnp.float32)
    total = jnp.sum(hist2)
    p = hist2 / total
    ent = -jnp.sum(p * jnp.log2(p + 1e-6))
    ent_ref[0] = jnp.full((8, 128), ent, jnp.float32)


def _tail_fused(t, w1, b1, w2, b2):
    n, c, himg, wimg = t.shape
    smem = pl.BlockSpec(memory_space=pltpu.SMEM)
    rec, ent = pl.pallas_call(
        _tail_kernel,
        grid=(n,),
        in_specs=[pl.BlockSpec((1, 3, himg, wimg), lambda i: (i, 0, 0, 0)),
                  smem, smem, smem, smem],
        out_specs=[pl.BlockSpec((1, 3, himg, wimg), lambda i: (i, 0, 0, 0)),
                   pl.BlockSpec((1, 8, 128), lambda i: (i, 0, 0))],
        out_shape=[jax.ShapeDtypeStruct((n, 3, himg, wimg), jnp.float32),
                   jax.ShapeDtypeStruct((n, 8, 128), jnp.float32)],
        scratch_shapes=[pltpu.VMEM((3, himg + 2 * _PAD, wimg), jnp.float32),
                        pltpu.VMEM((3, himg + 2 * _PAD, wimg), jnp.float32)],
        compiler_params=pltpu.CompilerParams(
            dimension_semantics=("parallel",)),
    )(t, w1.reshape(-1), b1, w2.reshape(-1), b2)
    return rec, ent[:, 0, 0]


def kernel(x, enc_params, dec_params):
    ep, dp = enc_params, dec_params
    h = _conv(x, ep['w0'], ep['b0'], 2)
    h = _gdn(h, ep['beta0'], ep['gamma0'])
    h = _conv(h, ep['w1'], ep['b1'], 2)
    h = _gdn(h, ep['beta1'], ep['gamma1'])
    h = _conv(h, ep['w2'], ep['b2'], 2)
    h = _gdn(h, ep['beta2'], ep['gamma2'])
    h = _conv(h, ep['w3'], ep['b3'], 2)
    latent = _resblock(h, ep['rw1'], ep['rb1'], ep['rw2'], ep['rb2'])

    b = latent + lax.stop_gradient(jnp.sign(latent) - latent)

    d = _deconv(b, dp['w0'], dp['b0'])
    d = _gdn(d, dp['beta0'], dp['gamma0'], inverse=True)
    d = _deconv(d, dp['w1'], dp['b1'])
    d = _gdn(d, dp['beta1'], dp['gamma1'], inverse=True)
    d = _deconv(d, dp['w2'], dp['b2'])
    d = _gdn(d, dp['beta2'], dp['gamma2'], inverse=True)
    d = _deconv(d, dp['w3'], dp['b3'])

    reconstructed, ent = _tail_fused(d, dp['rw1'], dp['rb1'],
                                     dp['rw2'], dp['rb2'])
    entropy = jnp.mean(ent)
    return reconstructed, latent, entropy


# speedup vs baseline: 1.0017x; 1.0017x over previous
import jax
import jax.numpy as jnp
from jax import lax
from jax.experimental import pallas as pl
from jax.experimental.pallas import tpu as pltpu

_DN = ('NCHW', 'OIHW', 'NCHW')


def _conv(x, w, b, stride=1):
    y = lax.conv_general_dilated(x, w, (stride, stride), ((1, 1), (1, 1)),
                                 dimension_numbers=_DN)
    return y + b[None, :, None, None]


def _deconv(x, w, b):
    y = lax.conv_general_dilated(x, w, (1, 1), ((1, 2), (1, 2)),
                                 lhs_dilation=(2, 2), dimension_numbers=_DN)
    return y + b[None, :, None, None]


def _gdn(x, beta, gamma, inverse=False):
    norm = jnp.sqrt(jnp.einsum('bihw,oi->bohw', x * x, gamma)
                    + beta[None, :, None, None])
    return x * norm if inverse else x / norm


def _resblock(x, w1, b1, w2, b2):
    h = jax.nn.relu(_conv(x, w1, b1, 1))
    return _conv(h, w2, b2, 1) + x


_PAD = 8
_NSTRIP = 4
_UNROLL = 8


def _shift_x(v, ox, w):
    if ox == 0:
        return v
    z = jnp.zeros((v.shape[0], 1), jnp.float32)
    if ox == 1:
        return jnp.concatenate([v[:, 1:], z], axis=1)
    return jnp.concatenate([z, v[:, :w - 1]], axis=1)


def _tail_kernel(t_ref, w1_ref, b1_ref, w2_ref, b2_ref,
                 rec_ref, ent_ref, pad_ref, h_ref):
    himg = rec_ref.shape[2]
    wimg = rec_ref.shape[3]
    strip = himg // _NSTRIP

    zband = jnp.zeros((_PAD, wimg), jnp.float32)
    for c in range(3):
        pad_ref[c, 0:_PAD, :] = zband
        pad_ref[c, _PAD + himg:, :] = zband
        pad_ref[c, _PAD:_PAD + himg, :] = t_ref[0, c]
        h_ref[c, 0:_PAD, :] = zband
        h_ref[c, _PAD + himg:, :] = zband

    def conv_strip(y0, src_read, wref, bref):
        outs = [jnp.full((strip, wimg), bref[co], jnp.float32)
                for co in range(3)]
        for ci in range(3):
            big = src_read(ci, y0)
            for dy in range(3):
                r = big[_PAD + dy - 1:_PAD + dy - 1 + strip]
                for dx in range(3):
                    t = _shift_x(r, dx - 1, wimg)
                    for co in range(3):
                        k = ((co * 3 + ci) * 3 + dy) * 3 + dx
                        outs[co] = outs[co] + wref[k] * t
        return outs

    def t_read(ci, y0):
        return pad_ref[ci, pl.ds(y0, strip + 2 * _PAD), :]

    def h_read(ci, y0):
        return h_ref[ci, pl.ds(y0, strip + 2 * _PAD), :]

    def conv1_body(s, carry):
        y0 = pl.multiple_of(s * strip, 8)
        outs = conv_strip(y0, t_read, w1_ref, b1_ref)
        for co in range(3):
            h_ref[co, pl.ds(y0 + _PAD, strip), :] = (
                jnp.maximum(outs[co], 0.0))
        return carry

    lax.fori_loop(0, _NSTRIP, conv1_body, 0)

    def conv2_body(s, carry):
        mn, mx = carry
        y0 = pl.multiple_of(s * strip, 8)
        outs = conv_strip(y0, h_read, w2_ref, b2_ref)
        for co in range(3):
            rec = jax.nn.sigmoid(
                outs[co] + t_ref[0, co, pl.ds(y0, strip), :])
            rec_ref[0, co, pl.ds(y0, strip), :] = rec
            mn = jnp.minimum(mn, jnp.min(rec))
            mx = jnp.maximum(mx, jnp.max(rec))
        return mn, mx

    mn, mx = lax.fori_loop(0, _NSTRIP, conv2_body,
                           (jnp.float32(jnp.inf), jnp.float32(-jnp.inf)))
    scale = jnp.where(mx > mn, 256.0 / (mx - mn), 0.0)

    iot = lax.broadcasted_iota(jnp.int32, (16, 8, wimg), 0)

    def one(c, j):
        chunk = rec_ref[0, c, pl.ds(j * 8, 8), :]
        idx = jnp.clip(jnp.floor((chunk - mn) * scale), 0.0, 255.0
                       ).astype(jnp.int32)
        hi = idx >> 4
        lo = idx & 15
        H = jnp.where(iot == hi[None], 1.0, 0.0).astype(jnp.bfloat16)
        L = jnp.where(iot == lo[None], 1.0, 0.0).astype(jnp.bfloat16)
        return lax.dot_general(H.reshape(128, wimg), L.reshape(128, wimg),
                               (((1,), (1,)), ((), ())),
                               preferred_element_type=jnp.float32)

    accs = tuple(jnp.zeros((128, 128), jnp.float32) for _ in range(_UNROLL))
    nchunk = himg // 8
    for c in range(3):
        def body(i, carry):
            return tuple(a + one(c, i * _UNROLL + u)
                         for u, a in enumerate(carry))
        accs = lax.fori_loop(0, nchunk // _UNROLL, body, accs)
    big = accs[0]
    for a in accs[1:]:
        big = big + a

    i_sub = lax.broadcasted_iota(jnp.int32, (128, 128), 0) & 7
    j_sub = lax.broadcasted_iota(jnp.int32, (128, 128), 1) & 7
    masked = jnp.where(i_sub == j_sub, big, 0.0)
    s = jnp.sum(masked.reshape(16, 8, 128), axis=1)
    gj = lax.broadcasted_iota(jnp.int32, (128, 16), 0) >> 3
    gl = lax.broadcasted_iota(jnp.int32, (128, 16), 1)
    G = jnp.where(gj == gl, 1.0, 0.0)
    hist2 = lax.dot_general(s, G, (((1,), (0,)), ((), ())),
                            preferred_element_type=jnp.float32)
    total = jnp.sum(hist2)
    p = hist2 / total
    ent = -jnp.sum(p * jnp.log2(p + 1e-6))
    ent_ref[0] = jnp.full((8, 128), ent, jnp.float32)


def _tail_fused(t, w1, b1, w2, b2):
    n, c, himg, wimg = t.shape
    smem = pl.BlockSpec(memory_space=pltpu.SMEM)
    rec, ent = pl.pallas_call(
        _tail_kernel,
        grid=(n,),
        in_specs=[pl.BlockSpec((1, 3, himg, wimg), lambda i: (i, 0, 0, 0)),
                  smem, smem, smem, smem],
        out_specs=[pl.BlockSpec((1, 3, himg, wimg), lambda i: (i, 0, 0, 0)),
                   pl.BlockSpec((1, 8, 128), lambda i: (i, 0, 0))],
        out_shape=[jax.ShapeDtypeStruct((n, 3, himg, wimg), jnp.float32),
                   jax.ShapeDtypeStruct((n, 8, 128), jnp.float32)],
        scratch_shapes=[pltpu.VMEM((3, himg + 2 * _PAD, wimg), jnp.float32),
                        pltpu.VMEM((3, himg + 2 * _PAD, wimg), jnp.float32)],
        compiler_params=pltpu.CompilerParams(
            dimension_semantics=("parallel",)),
    )(t, w1.reshape(-1), b1, w2.reshape(-1), b2)
    return rec, ent[:, 0, 0]


def kernel(x, enc_params, dec_params):
    ep, dp = enc_params, dec_params
    h = _conv(x, ep['w0'], ep['b0'], 2)
    h = _gdn(h, ep['beta0'], ep['gamma0'])
    h = _conv(h, ep['w1'], ep['b1'], 2)
    h = _gdn(h, ep['beta1'], ep['gamma1'])
    h = _conv(h, ep['w2'], ep['b2'], 2)
    h = _gdn(h, ep['beta2'], ep['gamma2'])
    h = _conv(h, ep['w3'], ep['b3'], 2)
    latent = _resblock(h, ep['rw1'], ep['rb1'], ep['rw2'], ep['rb2'])

    b = latent + lax.stop_gradient(jnp.sign(latent) - latent)

    d = _deconv(b, dp['w0'], dp['b0'])
    d = _gdn(d, dp['beta0'], dp['gamma0'], inverse=True)
    d = _deconv(d, dp['w1'], dp['b1'])
    d = _gdn(d, dp['beta1'], dp['gamma1'], inverse=True)
    d = _deconv(d, dp['w2'], dp['b2'])
    d = _gdn(d, dp['beta2'], dp['gamma2'], inverse=True)
    d = _deconv(d, dp['w3'], dp['b3'])

    reconstructed, ent = _tail_fused(d, dp['rw1'], dp['rb1'],
                                     dp['rw2'], dp['rb2'])
    entropy = jnp.mean(ent)
    return reconstructed, latent, entropy
